# baseline (speedup 1.0000x reference)
import functools

import numpy as np
import jax
import jax.numpy as jnp
from jax import lax
from jax.experimental import pallas as pl
from jax.experimental.pallas import tpu as pltpu
from jax.experimental.pallas import tpu_sc as plsc

_Y = 1024
_X = 4094
_NVEC = (512 + (np.arange(_Y) * 37) % 3584).astype(np.int64)
_TOTAL = int(_NVEC.sum())

_CK = tuple(int(-(-3584 * k // 37)) for k in range(1, 11))

_NW = 32
_ROWS_PER_W = _Y // _NW
_XPAD = 4096
_CH = 1024
_NCHI = 5
_NCHO = 4
_INBUF = _NCHI * _CH
_PAD = _INBUF + 128

_mesh = plsc.VectorSubcoreMesh(core_axis_name="c", subcore_axis_name="s")


def _row_params(r):
    t = 37 * r
    nv = 512 + lax.rem(t, 3584)
    tri = (t * (r - 1)) // 2
    tcount = 0
    for ck in _CK:
        tcount = tcount + lax.max(0, r - ck)
    off = 512 * r + tri - 3584 * tcount
    sh = lax.rem(off, 8)
    a = pl.multiple_of(off - sh, 8)
    return a, sh, nv


_NHALF = 1
_HROWS = _Y // _NHALF
_HROWS_PER_W = _HROWS // _NW
_DEPTH = 3


def _make_sc_scatter(base):
    @functools.partial(
        pl.kernel,
        mesh=_mesh,
        out_type=jax.ShapeDtypeStruct((_HROWS, _XPAD), jnp.float32),
        scratch_types=(
            [pltpu.VMEM((_INBUF,), jnp.float32)] * _DEPTH
            + [pltpu.VMEM((_XPAD,), jnp.float32)] * _DEPTH
            + [pltpu.SemaphoreType.DMA] * (2 * _DEPTH)
        ),
    )
    def _sc_scatter(logits_hbm, grid_hbm, *bufs):
        inbufs = bufs[0:_DEPTH]
        outbufs = bufs[_DEPTH:2 * _DEPTH]
        sis = bufs[2 * _DEPTH:3 * _DEPTH]
        sos = bufs[3 * _DEPTH:4 * _DEPTH]
        wid = lax.axis_index("s") * 2 + lax.axis_index("c")
        r0 = wid * _HROWS_PER_W

        params = [_row_params(base + r0 + j) for j in range(_HROWS_PER_W)]

        def issue_in(j, b):
            a, sh, nv = params[j]
            nchi = (sh + nv + _CH - 1) // _CH
            for c in range(_NCHI):
                @pl.when(c < nchi)
                def _():
                    pltpu.async_copy(
                        logits_hbm.at[pl.ds(pl.multiple_of(a + _CH * c, 8),
                                            _CH)],
                        inbufs[b].at[pl.ds(_CH * c, _CH)], sis[b])

        def wait_in(j, b):
            _, sh, nv = params[j]
            nchi = (sh + nv + _CH - 1) // _CH
            for c in range(_NCHI):
                @pl.when(c < nchi)
                def _():
                    pltpu.make_async_copy(
                        logits_hbm.at[pl.ds(0, _CH)],
                        inbufs[b].at[pl.ds(0, _CH)], sis[b]).wait()

        def issue_out(j, b):
            _, _, nv = params[j]
            ncho = (nv + _CH - 1) // _CH
            for c in range(_NCHO):
                @pl.when(c < ncho)
                def _():
                    pltpu.async_copy(
                        outbufs[b].at[pl.ds(_CH * c, _CH)],
                        grid_hbm.at[r0 + j, pl.ds(_CH * c, _CH)], sos[b])

        def wait_out(j, b):
            _, _, nv = params[j]
            ncho = (nv + _CH - 1) // _CH
            for c in range(_NCHO):
                @pl.when(c < ncho)
                def _():
                    pltpu.make_async_copy(
                        logits_hbm.at[pl.ds(0, _CH)],
                        outbufs[b].at[pl.ds(0, _CH)], sos[b]).wait()

        for j in range(_DEPTH):
            issue_in(j, j)

        for j in range(_HROWS_PER_W):
            b = j % _DEPTH
            inbuf, outbuf = inbufs[b], outbufs[b]
            a, sh, nv = params[j]
            nv16 = (nv + 15) // 16
            wait_in(j, b)
            if j >= _DEPTH:
                wait_out(j - _DEPTH, b)

            def copy_vec(c, carry, inbuf=inbuf, outbuf=outbuf, sh=sh):
                outbuf[pl.ds(16 * c, 16)] = inbuf[pl.ds(sh + 16 * c, 16)]
                return carry

            def copy_vec4(c, carry, inbuf=inbuf, outbuf=outbuf, sh=sh):
                for k in range(4):
                    o = 64 * c + 16 * k
                    outbuf[pl.ds(o, 16)] = inbuf[pl.ds(sh + o, 16)]
                return carry

            n4 = nv16 // 4
            lax.fori_loop(0, n4, copy_vec4, 0)
            lax.fori_loop(4 * n4, nv16, copy_vec, 0)

            issue_out(j, b)
            if j + _DEPTH < _HROWS_PER_W:
                issue_in(j + _DEPTH, b)

        for j in range(_HROWS_PER_W - _DEPTH, _HROWS_PER_W):
            wait_out(j, j % _DEPTH)

    return _sc_scatter


_sc_scatter_halves = [_make_sc_scatter(h * _HROWS) for h in range(_NHALF)]


def _lsm_body(gw_ref, nv_ref, g_ref, lp_ref):
    g_raw = gw_ref[...]
    nv = nv_ref[...]
    col = lax.broadcasted_iota(jnp.int32, g_raw.shape, 1)
    g = jnp.where(col < nv, g_raw, -jnp.inf)
    m = jnp.max(g, axis=-1, keepdims=True)
    e = jnp.exp(g - m)
    s = jnp.sum(e, axis=-1, keepdims=True)
    lp = g - (m + jnp.log(s))
    g_ref[...] = g[:, :_X]
    lp_ref[...] = lp[:, :_X]


_BR = 256
_NVEC_COL = _NVEC.astype(np.int32).reshape(_Y, 1)


def _lsm_first(gw_ref, nv_ref, g_ref, lp_ref):
    _lsm_body(gw_ref, nv_ref, g_ref, lp_ref)


def _lsm_next(gw_ref, nv_ref, g_in_ref, lp_in_ref, g_ref, lp_ref):
    del g_in_ref, lp_in_ref
    _lsm_body(gw_ref, nv_ref, g_ref, lp_ref)


def _log_softmax_half(h, grid_wide_h, nv_col, prev=None):
    boff = h * (_HROWS // _BR)
    out_specs = [
        pl.BlockSpec((_BR, _X), lambda i: (i + boff, 0)),
        pl.BlockSpec((_BR, _X), lambda i: (i + boff, 0)),
    ]
    out_shape = [
        jax.ShapeDtypeStruct((_Y, _X), jnp.float32),
        jax.ShapeDtypeStruct((_Y, _X), jnp.float32),
    ]
    in_specs = [
        pl.BlockSpec((_BR, _XPAD), lambda i: (i, 0)),
        pl.BlockSpec((_BR, 1), lambda i: (i + boff, 0)),
    ]
    params = pltpu.CompilerParams(dimension_semantics=("parallel",))
    if prev is None:
        return pl.pallas_call(
            _lsm_first,
            grid=(_HROWS // _BR,),
            in_specs=in_specs,
            out_specs=out_specs,
            out_shape=out_shape,
            compiler_params=params,
        )(grid_wide_h, nv_col)
    in_specs = in_specs + [
        pl.BlockSpec(memory_space=pl.ANY),
        pl.BlockSpec(memory_space=pl.ANY),
    ]
    return pl.pallas_call(
        _lsm_next,
        grid=(_HROWS // _BR,),
        in_specs=in_specs,
        out_specs=out_specs,
        out_shape=out_shape,
        input_output_aliases={2: 0, 3: 1},
        compiler_params=params,
    )(grid_wide_h, nv_col, prev[0], prev[1])


def kernel(logits):
    logits_pad = jnp.concatenate(
        [logits, jnp.zeros((_PAD,), jnp.float32)])
    nv_col = jnp.asarray(_NVEC_COL)
    gw = [f(logits_pad) for f in _sc_scatter_halves]
    outs = _log_softmax_half(0, gw[0], nv_col)
    for h in range(1, _NHALF):
        outs = _log_softmax_half(h, gw[h], nv_col, prev=outs)
    return (outs[0], outs[1])


# speedup vs baseline: 1.0886x; 1.0886x over previous
import functools

import numpy as np
import jax
import jax.numpy as jnp
from jax import lax
from jax.experimental import pallas as pl
from jax.experimental.pallas import tpu as pltpu
from jax.experimental.pallas import tpu_sc as plsc

_Y = 1024
_X = 4094
_NVEC = (512 + (np.arange(_Y) * 37) % 3584).astype(np.int64)
_TOTAL = int(_NVEC.sum())

_CK = tuple(int(-(-3584 * k // 37)) for k in range(1, 11))

_NW = 32
_ROWS_PER_W = _Y // _NW
_XPAD = 4096
_CH = 2048
_NCHI = 3
_NCHO = 2
_INBUF = _NCHI * _CH
_PAD = _INBUF + 128

_mesh = plsc.VectorSubcoreMesh(core_axis_name="c", subcore_axis_name="s")


def _row_params(r):
    t = 37 * r
    nv = 512 + lax.rem(t, 3584)
    tri = (t * (r - 1)) // 2
    tcount = 0
    for ck in _CK:
        tcount = tcount + lax.max(0, r - ck)
    off = 512 * r + tri - 3584 * tcount
    sh = lax.rem(off, 8)
    a = pl.multiple_of(off - sh, 8)
    return a, sh, nv


_NHALF = 1
_HROWS = _Y // _NHALF
_HROWS_PER_W = _HROWS // _NW
_DEPTH = 3


def _make_sc_scatter(base):
    @functools.partial(
        pl.kernel,
        mesh=_mesh,
        out_type=jax.ShapeDtypeStruct((_HROWS, _XPAD), jnp.float32),
        scratch_types=(
            [pltpu.VMEM((_INBUF,), jnp.float32)] * _DEPTH
            + [pltpu.VMEM((_XPAD,), jnp.float32)] * _DEPTH
            + [pltpu.SemaphoreType.DMA] * (2 * _DEPTH)
        ),
    )
    def _sc_scatter(logits_hbm, grid_hbm, *bufs):
        inbufs = bufs[0:_DEPTH]
        outbufs = bufs[_DEPTH:2 * _DEPTH]
        sis = bufs[2 * _DEPTH:3 * _DEPTH]
        sos = bufs[3 * _DEPTH:4 * _DEPTH]
        wid = lax.axis_index("s") * 2 + lax.axis_index("c")
        r0 = wid * _HROWS_PER_W

        params = [_row_params(base + r0 + j) for j in range(_HROWS_PER_W)]

        def issue_in(j, b):
            a, sh, nv = params[j]
            nchi = (sh + nv + _CH - 1) // _CH
            for c in range(_NCHI):
                @pl.when(c < nchi)
                def _():
                    pltpu.async_copy(
                        logits_hbm.at[pl.ds(pl.multiple_of(a + _CH * c, 8),
                                            _CH)],
                        inbufs[b].at[pl.ds(_CH * c, _CH)], sis[b])

        def wait_in(j, b):
            _, sh, nv = params[j]
            nchi = (sh + nv + _CH - 1) // _CH
            for c in range(_NCHI):
                @pl.when(c < nchi)
                def _():
                    pltpu.make_async_copy(
                        logits_hbm.at[pl.ds(0, _CH)],
                        inbufs[b].at[pl.ds(0, _CH)], sis[b]).wait()

        def issue_out(j, b):
            _, _, nv = params[j]
            ncho = (nv + _CH - 1) // _CH
            for c in range(_NCHO):
                @pl.when(c < ncho)
                def _():
                    pltpu.async_copy(
                        outbufs[b].at[pl.ds(_CH * c, _CH)],
                        grid_hbm.at[r0 + j, pl.ds(_CH * c, _CH)], sos[b])

        def wait_out(j, b):
            _, _, nv = params[j]
            ncho = (nv + _CH - 1) // _CH
            for c in range(_NCHO):
                @pl.when(c < ncho)
                def _():
                    pltpu.make_async_copy(
                        logits_hbm.at[pl.ds(0, _CH)],
                        outbufs[b].at[pl.ds(0, _CH)], sos[b]).wait()

        for j in range(_DEPTH):
            issue_in(j, j)

        for j in range(_HROWS_PER_W):
            b = j % _DEPTH
            inbuf, outbuf = inbufs[b], outbufs[b]
            a, sh, nv = params[j]
            nv16 = (nv + 15) // 16
            wait_in(j, b)
            if j >= _DEPTH:
                wait_out(j - _DEPTH, b)

            def copy_vec(c, carry, inbuf=inbuf, outbuf=outbuf, sh=sh):
                outbuf[pl.ds(16 * c, 16)] = inbuf[pl.ds(sh + 16 * c, 16)]
                return carry

            def copy_vec4(c, carry, inbuf=inbuf, outbuf=outbuf, sh=sh):
                for k in range(4):
                    o = 64 * c + 16 * k
                    outbuf[pl.ds(o, 16)] = inbuf[pl.ds(sh + o, 16)]
                return carry

            n4 = nv16 // 4
            lax.fori_loop(0, n4, copy_vec4, 0)
            lax.fori_loop(4 * n4, nv16, copy_vec, 0)

            issue_out(j, b)
            if j + _DEPTH < _HROWS_PER_W:
                issue_in(j + _DEPTH, b)

        for j in range(_HROWS_PER_W - _DEPTH, _HROWS_PER_W):
            wait_out(j, j % _DEPTH)

    return _sc_scatter


_sc_scatter_halves = [_make_sc_scatter(h * _HROWS) for h in range(_NHALF)]


def _lsm_body(gw_ref, nv_ref, g_ref, lp_ref):
    g_raw = gw_ref[...]
    nv = nv_ref[...]
    col = lax.broadcasted_iota(jnp.int32, g_raw.shape, 1)
    g = jnp.where(col < nv, g_raw, -jnp.inf)
    m = jnp.max(g, axis=-1, keepdims=True)
    e = jnp.exp(g - m)
    s = jnp.sum(e, axis=-1, keepdims=True)
    lp = g - (m + jnp.log(s))
    g_ref[...] = g[:, :_X]
    lp_ref[...] = lp[:, :_X]


_BR = 256
_NVEC_COL = _NVEC.astype(np.int32).reshape(_Y, 1)


def _lsm_first(gw_ref, nv_ref, g_ref, lp_ref):
    _lsm_body(gw_ref, nv_ref, g_ref, lp_ref)


def _lsm_next(gw_ref, nv_ref, g_in_ref, lp_in_ref, g_ref, lp_ref):
    del g_in_ref, lp_in_ref
    _lsm_body(gw_ref, nv_ref, g_ref, lp_ref)


def _log_softmax_half(h, grid_wide_h, nv_col, prev=None):
    boff = h * (_HROWS // _BR)
    out_specs = [
        pl.BlockSpec((_BR, _X), lambda i: (i + boff, 0)),
        pl.BlockSpec((_BR, _X), lambda i: (i + boff, 0)),
    ]
    out_shape = [
        jax.ShapeDtypeStruct((_Y, _X), jnp.float32),
        jax.ShapeDtypeStruct((_Y, _X), jnp.float32),
    ]
    in_specs = [
        pl.BlockSpec((_BR, _XPAD), lambda i: (i, 0)),
        pl.BlockSpec((_BR, 1), lambda i: (i + boff, 0)),
    ]
    if prev is None:
        return pl.pallas_call(
            _lsm_first,
            grid=(_HROWS // _BR,),
            in_specs=in_specs,
            out_specs=out_specs,
            out_shape=out_shape,
        )(grid_wide_h, nv_col)
    in_specs = in_specs + [
        pl.BlockSpec(memory_space=pl.ANY),
        pl.BlockSpec(memory_space=pl.ANY),
    ]
    return pl.pallas_call(
        _lsm_next,
        grid=(_HROWS // _BR,),
        in_specs=in_specs,
        out_specs=out_specs,
        out_shape=out_shape,
        input_output_aliases={2: 0, 3: 1},
    )(grid_wide_h, nv_col, prev[0], prev[1])


def kernel(logits):
    logits_pad = jnp.concatenate(
        [logits, jnp.zeros((_PAD,), jnp.float32)])
    nv_col = jnp.asarray(_NVEC_COL)
    gw = [f(logits_pad) for f in _sc_scatter_halves]
    outs = _log_softmax_half(0, gw[0], nv_col)
    for h in range(1, _NHALF):
        outs = _log_softmax_half(h, gw[h], nv_col, prev=outs)
    return (outs[0], outs[1])


# speedup vs baseline: 1.0981x; 1.0087x over previous
import functools

import numpy as np
import jax
import jax.numpy as jnp
from jax import lax
from jax.experimental import pallas as pl
from jax.experimental.pallas import tpu as pltpu
from jax.experimental.pallas import tpu_sc as plsc

_Y = 1024
_X = 4094
_NVEC = (512 + (np.arange(_Y) * 37) % 3584).astype(np.int64)
_TOTAL = int(_NVEC.sum())

_CK = tuple(int(-(-3584 * k // 37)) for k in range(1, 11))

_NW = 32
_ROWS_PER_W = _Y // _NW
_XPAD = 4096
_CH = 2048
_NCHI = 3
_NCHO = 1
_INBUF = _NCHI * _CH
_PAD = _INBUF + 128

_mesh = plsc.VectorSubcoreMesh(core_axis_name="c", subcore_axis_name="s")


def _row_params(r):
    t = 37 * r
    nv = 512 + lax.rem(t, 3584)
    tri = (t * (r - 1)) // 2
    tcount = 0
    for ck in _CK:
        tcount = tcount + lax.max(0, r - ck)
    off = 512 * r + tri - 3584 * tcount
    sh = lax.rem(off, 8)
    a = pl.multiple_of(off - sh, 8)
    return a, sh, nv


_NHALF = 1
_HROWS = _Y // _NHALF
_HROWS_PER_W = _HROWS // _NW
_DEPTH = 3


def _make_sc_scatter(base):
    @functools.partial(
        pl.kernel,
        mesh=_mesh,
        out_type=jax.ShapeDtypeStruct((_HROWS, _XPAD), jnp.float32),
        scratch_types=(
            [pltpu.VMEM((_INBUF,), jnp.float32)] * _DEPTH
            + [pltpu.VMEM((_XPAD,), jnp.float32)] * _DEPTH
            + [pltpu.SemaphoreType.DMA] * (2 * _DEPTH)
        ),
    )
    def _sc_scatter(logits_hbm, grid_hbm, *bufs):
        inbufs = bufs[0:_DEPTH]
        outbufs = bufs[_DEPTH:2 * _DEPTH]
        sis = bufs[2 * _DEPTH:3 * _DEPTH]
        sos = bufs[3 * _DEPTH:4 * _DEPTH]
        wid = lax.axis_index("s") * 2 + lax.axis_index("c")
        r0 = wid * _HROWS_PER_W

        params = [_row_params(base + r0 + j) for j in range(_HROWS_PER_W)]

        def issue_in(j, b):
            a, sh, nv = params[j]
            nchi = (sh + nv + _CH - 1) // _CH
            for c in range(_NCHI):
                @pl.when(c < nchi)
                def _():
                    pltpu.async_copy(
                        logits_hbm.at[pl.ds(pl.multiple_of(a + _CH * c, 8),
                                            _CH)],
                        inbufs[b].at[pl.ds(_CH * c, _CH)], sis[b])

        def wait_in(j, b):
            _, sh, nv = params[j]
            nchi = (sh + nv + _CH - 1) // _CH
            for c in range(_NCHI):
                @pl.when(c < nchi)
                def _():
                    pltpu.make_async_copy(
                        logits_hbm.at[pl.ds(0, _CH)],
                        inbufs[b].at[pl.ds(0, _CH)], sis[b]).wait()

        def issue_out(j, b):
            pltpu.async_copy(
                outbufs[b].at[pl.ds(0, _XPAD)],
                grid_hbm.at[r0 + j, pl.ds(0, _XPAD)], sos[b])

        def wait_out(j, b):
            pltpu.make_async_copy(
                logits_hbm.at[pl.ds(0, _XPAD)],
                outbufs[b].at[pl.ds(0, _XPAD)], sos[b]).wait()

        for j in range(_DEPTH):
            issue_in(j, j)

        for j in range(_HROWS_PER_W):
            b = j % _DEPTH
            inbuf, outbuf = inbufs[b], outbufs[b]
            a, sh, nv = params[j]
            nv16 = (nv + 15) // 16
            wait_in(j, b)
            if j >= _DEPTH:
                wait_out(j - _DEPTH, b)

            def copy_vec(c, carry, inbuf=inbuf, outbuf=outbuf, sh=sh):
                outbuf[pl.ds(16 * c, 16)] = inbuf[pl.ds(sh + 16 * c, 16)]
                return carry

            def copy_vec4(c, carry, inbuf=inbuf, outbuf=outbuf, sh=sh):
                for k in range(4):
                    o = 64 * c + 16 * k
                    outbuf[pl.ds(o, 16)] = inbuf[pl.ds(sh + o, 16)]
                return carry

            n4 = nv16 // 4
            lax.fori_loop(0, n4, copy_vec4, 0)
            lax.fori_loop(4 * n4, nv16, copy_vec, 0)

            issue_out(j, b)
            if j + _DEPTH < _HROWS_PER_W:
                issue_in(j + _DEPTH, b)

        for j in range(_HROWS_PER_W - _DEPTH, _HROWS_PER_W):
            wait_out(j, j % _DEPTH)

    return _sc_scatter


_sc_scatter_halves = [_make_sc_scatter(h * _HROWS) for h in range(_NHALF)]


def _lsm_body(gw_ref, nv_ref, g_ref, lp_ref):
    g_raw = gw_ref[...]
    nv = nv_ref[...]
    col = lax.broadcasted_iota(jnp.int32, g_raw.shape, 1)
    g = jnp.where(col < nv, g_raw, -jnp.inf)
    m = jnp.max(g, axis=-1, keepdims=True)
    e = jnp.exp(g - m)
    s = jnp.sum(e, axis=-1, keepdims=True)
    lp = g - (m + jnp.log(s))
    g_ref[...] = g[:, :_X]
    lp_ref[...] = lp[:, :_X]


_BR = 256
_NVEC_COL = _NVEC.astype(np.int32).reshape(_Y, 1)


def _lsm_first(gw_ref, nv_ref, g_ref, lp_ref):
    _lsm_body(gw_ref, nv_ref, g_ref, lp_ref)


def _lsm_next(gw_ref, nv_ref, g_in_ref, lp_in_ref, g_ref, lp_ref):
    del g_in_ref, lp_in_ref
    _lsm_body(gw_ref, nv_ref, g_ref, lp_ref)


def _log_softmax_half(h, grid_wide_h, nv_col, prev=None):
    boff = h * (_HROWS // _BR)
    out_specs = [
        pl.BlockSpec((_BR, _X), lambda i: (i + boff, 0)),
        pl.BlockSpec((_BR, _X), lambda i: (i + boff, 0)),
    ]
    out_shape = [
        jax.ShapeDtypeStruct((_Y, _X), jnp.float32),
        jax.ShapeDtypeStruct((_Y, _X), jnp.float32),
    ]
    in_specs = [
        pl.BlockSpec((_BR, _XPAD), lambda i: (i, 0)),
        pl.BlockSpec((_BR, 1), lambda i: (i + boff, 0)),
    ]
    if prev is None:
        return pl.pallas_call(
            _lsm_first,
            grid=(_HROWS // _BR,),
            in_specs=in_specs,
            out_specs=out_specs,
            out_shape=out_shape,
        )(grid_wide_h, nv_col)
    in_specs = in_specs + [
        pl.BlockSpec(memory_space=pl.ANY),
        pl.BlockSpec(memory_space=pl.ANY),
    ]
    return pl.pallas_call(
        _lsm_next,
        grid=(_HROWS // _BR,),
        in_specs=in_specs,
        out_specs=out_specs,
        out_shape=out_shape,
        input_output_aliases={2: 0, 3: 1},
    )(grid_wide_h, nv_col, prev[0], prev[1])


def kernel(logits):
    logits_pad = jnp.concatenate(
        [logits, jnp.zeros((_PAD,), jnp.float32)])
    nv_col = jnp.asarray(_NVEC_COL)
    gw = [f(logits_pad) for f in _sc_scatter_halves]
    outs = _log_softmax_half(0, gw[0], nv_col)
    for h in range(1, _NHALF):
        outs = _log_softmax_half(h, gw[h], nv_col, prev=outs)
    return (outs[0], outs[1])


# speedup vs baseline: 1.1166x; 1.0168x over previous
import functools

import numpy as np
import jax
import jax.numpy as jnp
from jax import lax
from jax.experimental import pallas as pl
from jax.experimental.pallas import tpu as pltpu
from jax.experimental.pallas import tpu_sc as plsc

_Y = 1024
_X = 4094
_NVEC = (512 + (np.arange(_Y) * 37) % 3584).astype(np.int64)
_TOTAL = int(_NVEC.sum())

_CK = tuple(int(-(-3584 * k // 37)) for k in range(1, 11))

_NW = 32
_ROWS_PER_W = _Y // _NW
_XPAD = 4096
_INBUF = 4160
_PAD = _INBUF + 128

_mesh = plsc.VectorSubcoreMesh(core_axis_name="c", subcore_axis_name="s")


def _row_params(r):
    t = 37 * r
    nv = 512 + lax.rem(t, 3584)
    tri = (t * (r - 1)) // 2
    tcount = 0
    for ck in _CK:
        tcount = tcount + lax.max(0, r - ck)
    off = 512 * r + tri - 3584 * tcount
    sh = lax.rem(off, 8)
    a = pl.multiple_of(off - sh, 8)
    return a, sh, nv


_NHALF = 1
_HROWS = _Y // _NHALF
_HROWS_PER_W = _HROWS // _NW
_DEPTH = 3


def _make_sc_scatter(base):
    @functools.partial(
        pl.kernel,
        mesh=_mesh,
        out_type=jax.ShapeDtypeStruct((_HROWS, _XPAD), jnp.float32),
        scratch_types=(
            [pltpu.VMEM((_INBUF,), jnp.float32)] * _DEPTH
            + [pltpu.VMEM((_XPAD,), jnp.float32)] * _DEPTH
            + [pltpu.SemaphoreType.DMA] * (2 * _DEPTH)
        ),
    )
    def _sc_scatter(logits_hbm, grid_hbm, *bufs):
        inbufs = bufs[0:_DEPTH]
        outbufs = bufs[_DEPTH:2 * _DEPTH]
        sis = bufs[2 * _DEPTH:3 * _DEPTH]
        sos = bufs[3 * _DEPTH:4 * _DEPTH]
        wid = lax.axis_index("s") * 2 + lax.axis_index("c")
        r0 = wid * _HROWS_PER_W

        params = [_row_params(base + r0 + j) for j in range(_HROWS_PER_W)]

        def issue_in(j, b):
            a, _, _ = params[j]
            pltpu.async_copy(
                logits_hbm.at[pl.ds(a, _INBUF)],
                inbufs[b].at[pl.ds(0, _INBUF)], sis[b])

        def wait_in(j, b):
            pltpu.make_async_copy(
                logits_hbm.at[pl.ds(0, _INBUF)],
                inbufs[b].at[pl.ds(0, _INBUF)], sis[b]).wait()

        def issue_out(j, b):
            pltpu.async_copy(
                outbufs[b].at[pl.ds(0, _XPAD)],
                grid_hbm.at[r0 + j, pl.ds(0, _XPAD)], sos[b])

        def wait_out(j, b):
            pltpu.make_async_copy(
                logits_hbm.at[pl.ds(0, _XPAD)],
                outbufs[b].at[pl.ds(0, _XPAD)], sos[b]).wait()

        for j in range(_DEPTH):
            issue_in(j, j)

        for j in range(_HROWS_PER_W):
            b = j % _DEPTH
            inbuf, outbuf = inbufs[b], outbufs[b]
            a, sh, nv = params[j]
            nv16 = (nv + 15) // 16
            wait_in(j, b)
            if j >= _DEPTH:
                wait_out(j - _DEPTH, b)

            def copy_vec(c, carry, inbuf=inbuf, outbuf=outbuf, sh=sh):
                outbuf[pl.ds(16 * c, 16)] = inbuf[pl.ds(sh + 16 * c, 16)]
                return carry

            def copy_vec4(c, carry, inbuf=inbuf, outbuf=outbuf, sh=sh):
                for k in range(4):
                    o = 64 * c + 16 * k
                    outbuf[pl.ds(o, 16)] = inbuf[pl.ds(sh + o, 16)]
                return carry

            n4 = nv16 // 4
            lax.fori_loop(0, n4, copy_vec4, 0)
            lax.fori_loop(4 * n4, nv16, copy_vec, 0)

            issue_out(j, b)
            if j + _DEPTH < _HROWS_PER_W:
                issue_in(j + _DEPTH, b)

        for j in range(_HROWS_PER_W - _DEPTH, _HROWS_PER_W):
            wait_out(j, j % _DEPTH)

    return _sc_scatter


_sc_scatter_halves = [_make_sc_scatter(h * _HROWS) for h in range(_NHALF)]


def _lsm_body(gw_ref, nv_ref, g_ref, lp_ref):
    g_raw = gw_ref[...]
    nv = nv_ref[...]
    col = lax.broadcasted_iota(jnp.int32, g_raw.shape, 1)
    g = jnp.where(col < nv, g_raw, -jnp.inf)
    m = jnp.max(g, axis=-1, keepdims=True)
    e = jnp.exp(g - m)
    s = jnp.sum(e, axis=-1, keepdims=True)
    lp = g - (m + jnp.log(s))
    g_ref[...] = g[:, :_X]
    lp_ref[...] = lp[:, :_X]


_BR = 256
_NVEC_COL = _NVEC.astype(np.int32).reshape(_Y, 1)


def _lsm_first(gw_ref, nv_ref, g_ref, lp_ref):
    _lsm_body(gw_ref, nv_ref, g_ref, lp_ref)


def _lsm_next(gw_ref, nv_ref, g_in_ref, lp_in_ref, g_ref, lp_ref):
    del g_in_ref, lp_in_ref
    _lsm_body(gw_ref, nv_ref, g_ref, lp_ref)


def _log_softmax_half(h, grid_wide_h, nv_col, prev=None):
    boff = h * (_HROWS // _BR)
    out_specs = [
        pl.BlockSpec((_BR, _X), lambda i: (i + boff, 0)),
        pl.BlockSpec((_BR, _X), lambda i: (i + boff, 0)),
    ]
    out_shape = [
        jax.ShapeDtypeStruct((_Y, _X), jnp.float32),
        jax.ShapeDtypeStruct((_Y, _X), jnp.float32),
    ]
    in_specs = [
        pl.BlockSpec((_BR, _XPAD), lambda i: (i, 0)),
        pl.BlockSpec((_BR, 1), lambda i: (i + boff, 0)),
    ]
    if prev is None:
        return pl.pallas_call(
            _lsm_first,
            grid=(_HROWS // _BR,),
            in_specs=in_specs,
            out_specs=out_specs,
            out_shape=out_shape,
        )(grid_wide_h, nv_col)
    in_specs = in_specs + [
        pl.BlockSpec(memory_space=pl.ANY),
        pl.BlockSpec(memory_space=pl.ANY),
    ]
    return pl.pallas_call(
        _lsm_next,
        grid=(_HROWS // _BR,),
        in_specs=in_specs,
        out_specs=out_specs,
        out_shape=out_shape,
        input_output_aliases={2: 0, 3: 1},
    )(grid_wide_h, nv_col, prev[0], prev[1])


def kernel(logits):
    logits_pad = jnp.concatenate(
        [logits, jnp.zeros((_PAD,), jnp.float32)])
    nv_col = jnp.asarray(_NVEC_COL)
    gw = [f(logits_pad) for f in _sc_scatter_halves]
    outs = _log_softmax_half(0, gw[0], nv_col)
    for h in range(1, _NHALF):
        outs = _log_softmax_half(h, gw[h], nv_col, prev=outs)
    return (outs[0], outs[1])
